# baseline (speedup 1.0000x reference)
import functools

import jax
import jax.numpy as jnp
from jax import lax
from jax.experimental import pallas as pl
from jax.experimental.pallas import tpu as pltpu
from jax.experimental.pallas import tpu_sc as plsc

N_TOKEN = 1000000
BATCH = 16384
HIST = 50
EMB = 64

CB = 1024
G1 = 489
QROWS = G1 * CB
NV = 2 * QROWS

NW = 32
HB = BATCH // 2
BW = HB // NW
GW = 128
C3 = 512


def _pair_transpose(tT):
    def body(xa_ref, xb_ref, o_ref):
        o_ref[:, 0:64] = xa_ref[...].T
        o_ref[:, 64:128] = xb_ref[...].T

    return pl.pallas_call(
        body,
        grid=(G1,),
        in_specs=[pl.BlockSpec((64, CB), lambda g: (0, 2 * g)),
                  pl.BlockSpec((64, CB), lambda g: (0, 2 * g + 1))],
        out_specs=pl.BlockSpec((CB, 128), lambda g: (g, 0)),
        out_shape=jax.ShapeDtypeStruct((QROWS, 128), jnp.float32),
    )(tT, tT)


def _sc_gather(j_inter, q):
    qr = q if q.ndim == 2 and q.shape[1] == EMB else q.reshape(NV, EMB)
    mesh = plsc.VectorSubcoreMesh(core_axis_name="c", subcore_axis_name="s")
    SUP = 2 * BW

    @functools.partial(
        pl.kernel,
        mesh=mesh,
        out_type=jax.ShapeDtypeStruct((HIST * BATCH, EMB), jnp.float32),
        compiler_params=pltpu.CompilerParams(use_tc_tiling_on_sc=False),
        scratch_types=[
            pltpu.VMEM((HIST * SUP,), jnp.int32),
            pltpu.VMEM((2, SUP, EMB), jnp.float32),
            pltpu.SemaphoreType.DMA,
            pltpu.SemaphoreType.DMA((2,)),
            pltpu.SemaphoreType.DMA((2,)),
        ],
    )
    def k(q_hbm, j_hbm, out_hbm, idx_v, rows_v, isem, gsem, wsem):
        wid = lax.axis_index("s") * 2 + lax.axis_index("c")
        base = 2 * BW * wid

        def idx_copy(h):
            return pltpu.make_async_copy(
                j_hbm.at[pl.ds(h * BATCH + base, SUP)],
                idx_v.at[pl.ds(h * SUP, SUP)], isem)

        def gather_copies(h, b):
            out = []
            for part in range(SUP // GW):
                out.append(pltpu.make_async_copy(
                    q_hbm.at[idx_v.at[pl.ds(h * SUP + part * GW, GW)]],
                    rows_v.at[b].at[pl.ds(part * GW, GW)],
                    gsem.at[b]))
            return out

        def fire_gathers(h, b):
            for c in gather_copies(h, b):
                c.start()

        def drain_gathers(h, b):
            for c in gather_copies(h, b):
                c.wait()

        def wb_copy(h, b):
            return pltpu.make_async_copy(
                rows_v.at[b],
                out_hbm.at[pl.ds(h * BATCH + base, SUP)], wsem.at[b])

        @pl.loop(0, HIST)
        def _(h):
            idx_copy(h).start()

        @pl.loop(0, HIST)
        def _(h):
            idx_copy(h).wait()

        fire_gathers(0, 0)

        @pl.loop(0, HIST, step=2)
        def _(g):
            for b in range(2):
                h = g + b
                oth = 1 - b
                nxt = h + 1

                @pl.when(nxt < HIST)
                def _():
                    @pl.when(h >= 1)
                    def _():
                        wb_copy(nxt - 2, oth).wait()

                    fire_gathers(nxt, oth)

                drain_gathers(h, b)
                wb_copy(h, b).start()

        wb_copy(HIST - 2, 0).wait()
        wb_copy(HIST - 1, 1).wait()

    return k(qr, j_inter)


def _untranspose(op5):
    def body(x_ref, o_ref):
        c = pl.program_id(1)
        o_ref[0, :, pl.ds(c * C3, C3)] = x_ref[:, 0:EMB].T
        o_ref[0, :, pl.ds(HB + c * C3, C3)] = x_ref[:, EMB:128].T

    return pl.pallas_call(
        body,
        grid=(HIST, HB // C3),
        in_specs=[pl.BlockSpec((C3, 128),
                               lambda h, c: (h * (HB // C3) + c, 0))],
        out_specs=pl.BlockSpec((1, EMB, BATCH), lambda h, c: (h, 0, 0)),
        out_shape=jax.ShapeDtypeStruct((HIST, EMB, BATCH), jnp.float32),
    )(op5)


def kernel(x, table):
    xT = x.astype(jnp.int32).T
    j = xT
    ji = j.reshape(HIST, 2, HB).transpose(0, 2, 1).reshape(HIST * BATCH)
    out6 = _sc_gather(ji, table)
    ot = _untranspose(out6.reshape(HIST * HB, 128))
    return ot.transpose(2, 0, 1)


# speedup vs baseline: 1.0550x; 1.0550x over previous
import functools

import jax
import jax.numpy as jnp
from jax import lax
from jax.experimental import pallas as pl
from jax.experimental.pallas import tpu as pltpu
from jax.experimental.pallas import tpu_sc as plsc

N_TOKEN = 1000000
BATCH = 16384
HIST = 50
EMB = 64

CB = 1024
G1 = 489
QROWS = G1 * CB
NV = 2 * QROWS

NW = 32
HB = BATCH // 2
BW = HB // NW
GW = 128
C3 = 512


def _pair_transpose(tT):
    def body(xa_ref, xb_ref, o_ref):
        o_ref[:, 0:64] = xa_ref[...].T
        o_ref[:, 64:128] = xb_ref[...].T

    return pl.pallas_call(
        body,
        grid=(G1,),
        in_specs=[pl.BlockSpec((64, CB), lambda g: (0, 2 * g)),
                  pl.BlockSpec((64, CB),
                               lambda g: (0, jnp.minimum(2 * g + 1,
                                                         2 * G1 - 3)))],
        out_specs=pl.BlockSpec((CB, 128), lambda g: (g, 0)),
        out_shape=jax.ShapeDtypeStruct((QROWS, 128), jnp.float32),
    )(tT, tT)


def _sc_gather(j_inter, q):
    qr = q if q.ndim == 2 and q.shape[1] == EMB else q.reshape(NV, EMB)
    mesh = plsc.VectorSubcoreMesh(core_axis_name="c", subcore_axis_name="s")
    SUP = 2 * BW

    @functools.partial(
        pl.kernel,
        mesh=mesh,
        out_type=jax.ShapeDtypeStruct((HIST * BATCH, EMB), jnp.float32),
        compiler_params=pltpu.CompilerParams(use_tc_tiling_on_sc=False),
        scratch_types=[
            pltpu.VMEM((HIST * SUP,), jnp.int32),
            pltpu.VMEM((2, SUP, EMB), jnp.float32),
            pltpu.SemaphoreType.DMA,
            pltpu.SemaphoreType.DMA((2,)),
            pltpu.SemaphoreType.DMA((2,)),
        ],
    )
    def k(q_hbm, j_hbm, out_hbm, idx_v, rows_v, isem, gsem, wsem):
        wid = lax.axis_index("s") * 2 + lax.axis_index("c")
        base = 2 * BW * wid

        def idx_copy(h):
            return pltpu.make_async_copy(
                j_hbm.at[pl.ds(h * BATCH + base, SUP)],
                idx_v.at[pl.ds(h * SUP, SUP)], isem)

        def gather_copies(h, b):
            out = []
            for part in range(SUP // GW):
                out.append(pltpu.make_async_copy(
                    q_hbm.at[idx_v.at[pl.ds(h * SUP + part * GW, GW)]],
                    rows_v.at[b].at[pl.ds(part * GW, GW)],
                    gsem.at[b]))
            return out

        def fire_gathers(h, b):
            for c in gather_copies(h, b):
                c.start()

        def drain_gathers(h, b):
            for c in gather_copies(h, b):
                c.wait()

        def wb_copy(h, b):
            return pltpu.make_async_copy(
                rows_v.at[b],
                out_hbm.at[pl.ds(h * BATCH + base, SUP)], wsem.at[b])

        @pl.loop(0, HIST)
        def _(h):
            idx_copy(h).start()

        @pl.loop(0, HIST)
        def _(h):
            idx_copy(h).wait()

        fire_gathers(0, 0)

        @pl.loop(0, HIST, step=2)
        def _(g):
            for b in range(2):
                h = g + b
                oth = 1 - b
                nxt = h + 1

                @pl.when(nxt < HIST)
                def _():
                    @pl.when(h >= 1)
                    def _():
                        wb_copy(nxt - 2, oth).wait()

                    fire_gathers(nxt, oth)

                drain_gathers(h, b)
                wb_copy(h, b).start()

        wb_copy(HIST - 2, 0).wait()
        wb_copy(HIST - 1, 1).wait()

    return k(qr, j_inter)


def _untranspose(op5):
    def body(x_ref, o_ref):
        c = pl.program_id(1)
        o_ref[0, :, pl.ds(c * C3, C3)] = x_ref[:, 0:EMB].T
        o_ref[0, :, pl.ds(HB + c * C3, C3)] = x_ref[:, EMB:128].T

    return pl.pallas_call(
        body,
        grid=(HIST, HB // C3),
        in_specs=[pl.BlockSpec((C3, 128),
                               lambda h, c: (h * (HB // C3) + c, 0))],
        out_specs=pl.BlockSpec((1, EMB, BATCH), lambda h, c: (h, 0, 0)),
        out_shape=jax.ShapeDtypeStruct((HIST, EMB, BATCH), jnp.float32),
    )(op5)


def kernel(x, table):
    xT = x.astype(jnp.int32).T
    m = xT & 2047
    j = (xT - m) + jnp.where(m < CB, m << 1, (m << 1) - (2 * CB - 1))
    ji = j.reshape(HIST, 2, HB).transpose(0, 2, 1).reshape(HIST * BATCH)
    q = _pair_transpose(table.T)
    out6 = _sc_gather(ji, q)
    ot = _untranspose(out6.reshape(HIST * HB, 128))
    return ot.transpose(2, 0, 1)


# speedup vs baseline: 1.3203x; 1.2514x over previous
import functools

import jax
import jax.numpy as jnp
from jax import lax
from jax.experimental import pallas as pl
from jax.experimental.pallas import tpu as pltpu
from jax.experimental.pallas import tpu_sc as plsc

N_TOKEN = 1000000
BATCH = 16384
HIST = 50
EMB = 64
NUM_IDX = BATCH * HIST

CB = 1024
G1 = 489
QROWS = G1 * CB
NV = 2 * QROWS

NW = 32
B_PER_W = NUM_IDX // NW
GW = 128
SUP = 512
KG = SUP // GW
N_SUP = B_PER_W // SUP


def _pair_transpose(tT):
    def body(xa_ref, xb_ref, o_ref):
        o_ref[:, 0:64] = xa_ref[...].T
        o_ref[:, 64:128] = xb_ref[...].T

    return pl.pallas_call(
        body,
        grid=(G1,),
        in_specs=[pl.BlockSpec((64, CB), lambda g: (0, 2 * g)),
                  pl.BlockSpec((64, CB),
                               lambda g: (0, jnp.minimum(2 * g + 1,
                                                         2 * G1 - 3)))],
        out_specs=pl.BlockSpec((CB, 128), lambda g: (g, 0)),
        out_shape=jax.ShapeDtypeStruct((QROWS, 128), jnp.float32),
    )(tT, tT)


def _sc_gather(idx, q):
    qr = q.reshape(NV, EMB)
    mesh = plsc.VectorSubcoreMesh(core_axis_name="c", subcore_axis_name="s")

    @functools.partial(
        pl.kernel,
        mesh=mesh,
        out_type=jax.ShapeDtypeStruct((NUM_IDX, EMB), jnp.float32),
        compiler_params=pltpu.CompilerParams(use_tc_tiling_on_sc=False),
        scratch_types=[
            pltpu.VMEM((B_PER_W,), jnp.int32),
            pltpu.VMEM((2, SUP, EMB), jnp.float32),
            pltpu.SemaphoreType.DMA((2,)),
            pltpu.SemaphoreType.DMA((2,)),
        ],
    )
    def k(q_hbm, idx_hbm, out_hbm, idx_v, rows_v, gsem, wsem):
        wid = lax.axis_index("s") * 2 + lax.axis_index("c")
        base = wid * B_PER_W

        pltpu.sync_copy(idx_hbm.at[pl.ds(base, B_PER_W)], idx_v)

        def fire(slot, b):
            for j in range(KG):
                pltpu.async_copy(
                    q_hbm.at[idx_v.at[pl.ds(slot * SUP + j * GW, GW)]],
                    rows_v.at[b].at[pl.ds(j * GW, GW)],
                    gsem.at[b],
                )

        def drain_gathers(slot, b):
            for j in range(KG):
                pltpu.make_async_copy(
                    q_hbm.at[idx_v.at[pl.ds(slot * SUP + j * GW, GW)]],
                    rows_v.at[b].at[pl.ds(j * GW, GW)],
                    gsem.at[b],
                ).wait()

        def wait_writeback(b):
            pltpu.make_async_copy(
                rows_v.at[b],
                out_hbm.at[pl.ds(base, SUP)],
                wsem.at[b],
            ).wait()

        fire(0, 0)

        @pl.loop(0, N_SUP, step=2)
        def _(g):
            for b in range(2):
                slot = g + b
                oth = 1 - b
                nxt = slot + 1

                @pl.when(nxt < N_SUP)
                def _():
                    @pl.when(slot >= 1)
                    def _():
                        wait_writeback(oth)

                    fire(nxt, oth)

                drain_gathers(slot, b)
                pltpu.async_copy(
                    rows_v.at[b],
                    out_hbm.at[pl.ds(base + slot * SUP, SUP)],
                    wsem.at[b],
                )

        for b in range(2):
            wait_writeback(b)

    return k(qr, idx)


def kernel(x, table):
    xi = x.astype(jnp.int32)
    m = xi & 2047
    j = (xi - m) + jnp.where(m < CB, m << 1, (m << 1) - (2 * CB - 1))
    q = _pair_transpose(table.T)
    out = _sc_gather(j.reshape(-1), q)
    return out.reshape(BATCH, HIST, EMB)


# speedup vs baseline: 1.3204x; 1.0001x over previous
import functools

import jax
import jax.numpy as jnp
from jax import lax
from jax.experimental import pallas as pl
from jax.experimental.pallas import tpu as pltpu
from jax.experimental.pallas import tpu_sc as plsc

N_TOKEN = 1000000
BATCH = 16384
HIST = 50
EMB = 64
NUM_IDX = BATCH * HIST

CB = 1024
G1 = 489
QROWS = G1 * CB
NV = 2 * QROWS

NW = 32
B_PER_W = NUM_IDX // NW
GW = 256
SUP = 512
KG = SUP // GW
N_SUP = B_PER_W // SUP


def _pair_transpose(tT):
    def body(xa_ref, xb_ref, o_ref):
        o_ref[:, 0:64] = xa_ref[...].T
        o_ref[:, 64:128] = xb_ref[...].T

    return pl.pallas_call(
        body,
        grid=(G1,),
        in_specs=[pl.BlockSpec((64, CB), lambda g: (0, 2 * g)),
                  pl.BlockSpec((64, CB),
                               lambda g: (0, jnp.minimum(2 * g + 1,
                                                         2 * G1 - 3)))],
        out_specs=pl.BlockSpec((CB, 128), lambda g: (g, 0)),
        out_shape=jax.ShapeDtypeStruct((QROWS, 128), jnp.float32),
    )(tT, tT)


def _sc_gather(idx, q):
    qr = q.reshape(NV, EMB)
    mesh = plsc.VectorSubcoreMesh(core_axis_name="c", subcore_axis_name="s")

    @functools.partial(
        pl.kernel,
        mesh=mesh,
        out_type=jax.ShapeDtypeStruct((NUM_IDX, EMB), jnp.float32),
        compiler_params=pltpu.CompilerParams(use_tc_tiling_on_sc=False),
        scratch_types=[
            pltpu.VMEM((B_PER_W,), jnp.int32),
            pltpu.VMEM((2, SUP, EMB), jnp.float32),
            pltpu.SemaphoreType.DMA((2,)),
            pltpu.SemaphoreType.DMA((2,)),
        ],
    )
    def k(q_hbm, idx_hbm, out_hbm, idx_v, rows_v, gsem, wsem):
        wid = lax.axis_index("s") * 2 + lax.axis_index("c")
        base = wid * B_PER_W

        pltpu.sync_copy(idx_hbm.at[pl.ds(base, B_PER_W)], idx_v)

        def fire(slot, b):
            for j in range(KG):
                pltpu.async_copy(
                    q_hbm.at[idx_v.at[pl.ds(slot * SUP + j * GW, GW)]],
                    rows_v.at[b].at[pl.ds(j * GW, GW)],
                    gsem.at[b],
                )

        def drain_gathers(slot, b):
            for j in range(KG):
                pltpu.make_async_copy(
                    q_hbm.at[idx_v.at[pl.ds(slot * SUP + j * GW, GW)]],
                    rows_v.at[b].at[pl.ds(j * GW, GW)],
                    gsem.at[b],
                ).wait()

        def wait_writeback(b):
            pltpu.make_async_copy(
                rows_v.at[b],
                out_hbm.at[pl.ds(base, SUP)],
                wsem.at[b],
            ).wait()

        fire(0, 0)

        @pl.loop(0, N_SUP, step=2)
        def _(g):
            for b in range(2):
                slot = g + b
                oth = 1 - b
                nxt = slot + 1

                @pl.when(nxt < N_SUP)
                def _():
                    @pl.when(slot >= 1)
                    def _():
                        wait_writeback(oth)

                    fire(nxt, oth)

                drain_gathers(slot, b)
                pltpu.async_copy(
                    rows_v.at[b],
                    out_hbm.at[pl.ds(base + slot * SUP, SUP)],
                    wsem.at[b],
                )

        for b in range(2):
            wait_writeback(b)

    return k(qr, idx)


def kernel(x, table):
    xi = x.astype(jnp.int32)
    m = xi & 2047
    j = (xi - m) + jnp.where(m < CB, m << 1, (m << 1) - (2 * CB - 1))
    q = _pair_transpose(table.T)
    out = _sc_gather(j.reshape(-1), q)
    return out.reshape(BATCH, HIST, EMB)
